# Initial kernel scaffold; baseline (speedup 1.0000x reference)
import jax
import jax.numpy as jnp
from jax.experimental import pallas as pl


def kernel(x_websites, x_users, ei_u2w, ei_w2u, lin_w_web, lin_b_web, lin_w_usr, lin_b_usr, c1_uw_l, c1_uw_r, c1_uw_b, c1_wu_l, c1_wu_r, c1_wu_b, c2_uw_l, c2_uw_r, c2_uw_b, c2_wu_l, c2_wu_r, c2_wu_b):
    raise NotImplementedError("write your pallas kernel here")


# speedup vs baseline: 3.4433x; 3.4433x over previous
import functools

import jax
import jax.numpy as jnp
from jax import lax
from jax.experimental import pallas as pl
from jax.experimental.pallas import tpu as pltpu
from jax.experimental.pallas import tpu_sc as plsc

_N = 10000
_D_IN = 128
_HID = 256
_OUT = 64
_E = 320000

_NT = 16
_NP = 10240
_RT = _NP // _NT
_B = 80
_ET = _E // _NT
_NCH = _ET // _B

_BN = 1000


def _dot(a, b):
    return jnp.dot(a, b, precision=lax.Precision.HIGHEST,
                   preferred_element_type=jnp.float32)


def _proj_body(x_ref, w_ref, b_ref, h0_ref, h1_ref):
    h = _dot(x_ref[...], w_ref[...]) + b_ref[...]
    h = jnp.maximum(h, 0.0)
    h0_ref[...] = h[:, :128]
    h1_ref[...] = h[:, 128:]


def _proj(x, w, b2):
    return pl.pallas_call(
        _proj_body,
        grid=(_N // _BN,),
        in_specs=[
            pl.BlockSpec((_BN, _D_IN), lambda i: (i, 0)),
            pl.BlockSpec((_D_IN, _HID), lambda i: (0, 0)),
            pl.BlockSpec((1, _HID), lambda i: (0, 0)),
        ],
        out_specs=[
            pl.BlockSpec((_BN, 128), lambda i: (i, 0)),
            pl.BlockSpec((_BN, 128), lambda i: (i, 0)),
        ],
        out_shape=[jax.ShapeDtypeStruct((_N, 128), jnp.float32)] * 2,
    )(x, w, b2)


def _edge_loop(s, src, dst, sidx, didx, acc, body_fn):
    def chunk(i, carry):
        e0 = pl.multiple_of(s * _ET + i * _B, _B)
        if src is not None:
            pltpu.sync_copy(src.at[pl.ds(e0, _B)], sidx)
        pltpu.sync_copy(dst.at[pl.ds(e0, _B)], didx.at[0])
        body_fn()
        return carry
    lax.fori_loop(0, _NCH, chunk, 0)


def _gather_scatter_dir(s, table, src, dst, sidx, didx, rows, acc):
    def body():
        pltpu.sync_copy(table.at[sidx], rows)
        pltpu.sync_copy(rows, acc.at[didx.at[0]], add=True)
    _edge_loop(s, src, dst, sidx, didx, acc, body)


def _count_dir(s, dst, sidx, didx, ones_v, acc):
    def body():
        pltpu.sync_copy(ones_v, acc.at[didx.at[0]], add=True)
    _edge_loop(s, None, dst, sidx, didx, acc, body)


def _zero_rows(zfeat, acc, r0):
    pltpu.sync_copy(zfeat.at[pl.ds(r0, _RT), :], acc.at[pl.ds(r0, _RT), :])


def _writeout(acc, out, r0):
    pltpu.sync_copy(acc.at[pl.ds(r0, _RT), :], out.at[pl.ds(r0, _RT), :])


def _agg1_body(hu0, hu1, hw0, hw1, s_uw, d_uw, s_wu, d_wu, zfeat, ones_h,
               sw0, sw1, su0, su1, cw, cu,
               acc, rows, sidx, didx, ones_v):
    c = lax.axis_index("c")
    s = lax.axis_index("s")
    r0 = s * _RT

    pltpu.sync_copy(ones_h, ones_v)
    _zero_rows(zfeat, acc, r0)
    plsc.subcore_barrier()

    @pl.when(c == 0)
    def _():
        _count_dir(s, d_uw, sidx, didx, ones_v, acc)

    @pl.when(c == 1)
    def _():
        _count_dir(s, d_wu, sidx, didx, ones_v, acc)

    plsc.subcore_barrier()

    @pl.when(c == 0)
    def _():
        _writeout(acc, cw, r0)

    @pl.when(c == 1)
    def _():
        _writeout(acc, cu, r0)

    _zero_rows(zfeat, acc, r0)
    plsc.subcore_barrier()

    @pl.when(c == 0)
    def _():
        _gather_scatter_dir(s, hu0, s_uw, d_uw, sidx, didx, rows, acc)

    @pl.when(c == 1)
    def _():
        _gather_scatter_dir(s, hu1, s_uw, d_uw, sidx, didx, rows, acc)

    plsc.subcore_barrier()

    @pl.when(c == 0)
    def _():
        _writeout(acc, sw0, r0)

    @pl.when(c == 1)
    def _():
        _writeout(acc, sw1, r0)

    _zero_rows(zfeat, acc, r0)
    plsc.subcore_barrier()

    @pl.when(c == 0)
    def _():
        _gather_scatter_dir(s, hw0, s_wu, d_wu, sidx, didx, rows, acc)

    @pl.when(c == 1)
    def _():
        _gather_scatter_dir(s, hw1, s_wu, d_wu, sidx, didx, rows, acc)

    plsc.subcore_barrier()

    @pl.when(c == 0)
    def _():
        _writeout(acc, su0, r0)

    @pl.when(c == 1)
    def _():
        _writeout(acc, su1, r0)


def _make_agg1():
    mesh = plsc.VectorSubcoreMesh(core_axis_name="c", subcore_axis_name="s")
    out_type = [jax.ShapeDtypeStruct((_NP, 128), jnp.float32)] * 6
    scratch = [
        pltpu.VMEM_SHARED((_NP, 128), jnp.float32),
        pltpu.VMEM((_B, 128), jnp.float32),
        pltpu.VMEM((_B,), jnp.int32),
        pltpu.VMEM((1, _B), jnp.int32),
        pltpu.VMEM((_B, 128), jnp.float32),
    ]
    return pl.kernel(
        _agg1_body,
        out_type=out_type,
        mesh=mesh,
        scratch_types=scratch,
    )


def _agg2_body(pu, pw, s_uw, d_uw, s_wu, d_wu, zfeat,
               s2w, s2u,
               acc, rows, sidx, didx):
    c = lax.axis_index("c")
    s = lax.axis_index("s")
    r0 = s * _RT

    _zero_rows(zfeat, acc, r0)
    plsc.subcore_barrier()

    @pl.when(c == 0)
    def _():
        _gather_scatter_dir(s, pu, s_uw, d_uw, sidx, didx, rows, acc)

    @pl.when(c == 1)
    def _():
        _gather_scatter_dir(s, pw, s_wu, d_wu, sidx, didx, rows, acc)

    plsc.subcore_barrier()

    @pl.when(c == 0)
    def _():
        _writeout(acc, s2w, r0)

    @pl.when(c == 1)
    def _():
        _writeout(acc, s2u, r0)


def _make_agg2():
    mesh = plsc.VectorSubcoreMesh(core_axis_name="c", subcore_axis_name="s")
    out_type = [jax.ShapeDtypeStruct((_NP, 128), jnp.float32)] * 2
    scratch = [
        pltpu.VMEM_SHARED((_NP, 128), jnp.float32),
        pltpu.VMEM((_B, 128), jnp.float32),
        pltpu.VMEM((_B,), jnp.int32),
        pltpu.VMEM((1, _B), jnp.int32),
    ]
    return pl.kernel(
        _agg2_body,
        out_type=out_type,
        mesh=mesh,
        scratch_types=scratch,
    )


def _comb_body(s0_ref, s1_ref, cnt_ref, h0_ref, h1_ref, l_ref, r_ref, b_ref,
               c2_ref, o_ref, p_ref):
    inv = 1.0 / jnp.maximum(cnt_ref[...][:, :1], 1.0)
    lw = l_ref[...]
    rw = r_ref[...]
    o = (_dot(s0_ref[...] * inv, lw[:128, :])
         + _dot(s1_ref[...] * inv, lw[128:, :])
         + _dot(h0_ref[...], rw[:128, :])
         + _dot(h1_ref[...], rw[128:, :])
         + b_ref[...])
    o = jnp.maximum(o, 0.0)
    o_ref[...] = o
    p = _dot(o, c2_ref[...])
    p_ref[...] = jnp.concatenate(
        [p, jnp.zeros((p.shape[0], 128 - _OUT), jnp.float32)], axis=1)


def _comb(s0, s1, cnt, h0, h1, lw, rw, b2, c2):
    return pl.pallas_call(
        _comb_body,
        grid=(_N // _BN,),
        in_specs=[
            pl.BlockSpec((_BN, 128), lambda i: (i, 0)),
            pl.BlockSpec((_BN, 128), lambda i: (i, 0)),
            pl.BlockSpec((_BN, 128), lambda i: (i, 0)),
            pl.BlockSpec((_BN, 128), lambda i: (i, 0)),
            pl.BlockSpec((_BN, 128), lambda i: (i, 0)),
            pl.BlockSpec((_HID, _HID), lambda i: (0, 0)),
            pl.BlockSpec((_HID, _HID), lambda i: (0, 0)),
            pl.BlockSpec((1, _HID), lambda i: (0, 0)),
            pl.BlockSpec((_HID, _OUT), lambda i: (0, 0)),
        ],
        out_specs=[
            pl.BlockSpec((_BN, _HID), lambda i: (i, 0)),
            pl.BlockSpec((_BN, 128), lambda i: (i, 0)),
        ],
        out_shape=[
            jax.ShapeDtypeStruct((_N, _HID), jnp.float32),
            jax.ShapeDtypeStruct((_N, 128), jnp.float32),
        ],
    )(s0, s1, cnt, h0, h1, lw, rw, b2, c2)


def _final_body(t_ref, cnt_ref, o_ref, r_ref, b_ref, z_ref):
    inv = 1.0 / jnp.maximum(cnt_ref[...][:, :1], 1.0)
    s2 = t_ref[...][:, :_OUT]
    z = s2 * inv + _dot(o_ref[...], r_ref[...]) + b_ref[...]
    m = jnp.max(z, axis=1, keepdims=True)
    ez = jnp.exp(z - m)
    lse = jnp.log(jnp.sum(ez, axis=1, keepdims=True))
    z_ref[...] = z - m - lse


def _final(t, cnt, o, rw, b2):
    return pl.pallas_call(
        _final_body,
        grid=(_N // _BN,),
        in_specs=[
            pl.BlockSpec((_BN, 128), lambda i: (i, 0)),
            pl.BlockSpec((_BN, 128), lambda i: (i, 0)),
            pl.BlockSpec((_BN, _HID), lambda i: (i, 0)),
            pl.BlockSpec((_HID, _OUT), lambda i: (0, 0)),
            pl.BlockSpec((1, _OUT), lambda i: (0, 0)),
        ],
        out_specs=pl.BlockSpec((_BN, _OUT), lambda i: (i, 0)),
        out_shape=jax.ShapeDtypeStruct((_N, _OUT), jnp.float32),
    )(t, cnt, o, rw, b2)


def kernel(x_websites, x_users, ei_u2w, ei_w2u,
           lin_w_web, lin_b_web, lin_w_usr, lin_b_usr,
           c1_uw_l, c1_uw_r, c1_uw_b, c1_wu_l, c1_wu_r, c1_wu_b,
           c2_uw_l, c2_uw_r, c2_uw_b, c2_wu_l, c2_wu_r, c2_wu_b):
    z128 = jnp.zeros((_NP, 128), jnp.float32)
    ones_h = jnp.ones((_B, 128), jnp.float32)
    s_uw, d_uw = ei_u2w[0], ei_u2w[1]
    s_wu, d_wu = ei_w2u[0], ei_w2u[1]

    h_w0, h_w1 = _proj(x_websites, lin_w_web, lin_b_web.reshape(1, -1))
    h_u0, h_u1 = _proj(x_users, lin_w_usr, lin_b_usr.reshape(1, -1))

    sw0, sw1, su0, su1, cw, cu = _make_agg1()(
        h_u0, h_u1, h_w0, h_w1, s_uw, d_uw, s_wu, d_wu, z128, ones_h)

    o_w, p_w = _comb(sw0, sw1, cw, h_w0, h_w1,
                     c1_uw_l, c1_uw_r, c1_uw_b.reshape(1, -1), c2_wu_l)
    o_u, p_u = _comb(su0, su1, cu, h_u0, h_u1,
                     c1_wu_l, c1_wu_r, c1_wu_b.reshape(1, -1), c2_uw_l)

    s2w, s2u = _make_agg2()(
        p_u, p_w, s_uw, d_uw, s_wu, d_wu, z128)

    z_w = _final(s2w, cw, o_w, c2_uw_r, c2_uw_b.reshape(1, -1))
    z_u = _final(s2u, cu, o_u, c2_wu_r, c2_wu_b.reshape(1, -1))
    return (z_w, z_u)


# speedup vs baseline: 5.1764x; 1.5034x over previous
import functools

import jax
import jax.numpy as jnp
from jax import lax
from jax.experimental import pallas as pl
from jax.experimental.pallas import tpu as pltpu
from jax.experimental.pallas import tpu_sc as plsc

_N = 10000
_D_IN = 128
_HID = 256
_OUT = 64
_E = 320000

_NT = 16
_NP = 10240
_RT = _NP // _NT
_B = 80
_ET = _E // _NT
_NCH = _ET // _B

_BN = 1000


def _dot(a, b):
    return jnp.dot(a, b, precision=lax.Precision.HIGHEST,
                   preferred_element_type=jnp.float32)


def _proj_body(x_ref, w_ref, b_ref, h0_ref, h1_ref):
    h = _dot(x_ref[...], w_ref[...]) + b_ref[...]
    h = jnp.maximum(h, 0.0)
    h0_ref[...] = h[:, :128]
    h1_ref[...] = h[:, 128:]


def _proj(x, w, b2):
    return pl.pallas_call(
        _proj_body,
        grid=(_N // _BN,),
        in_specs=[
            pl.BlockSpec((_BN, _D_IN), lambda i: (i, 0)),
            pl.BlockSpec((_D_IN, _HID), lambda i: (0, 0)),
            pl.BlockSpec((1, _HID), lambda i: (0, 0)),
        ],
        out_specs=[
            pl.BlockSpec((_BN, 128), lambda i: (i, 0)),
            pl.BlockSpec((_BN, 128), lambda i: (i, 0)),
        ],
        out_shape=[jax.ShapeDtypeStruct((_N, 128), jnp.float32)] * 2,
    )(x, w, b2)


def _gather_scatter_dir(s, table, src, dst, sidx, didx, rows, acc, sems):
    def stage(ch, b):
        e0 = pl.multiple_of(s * _ET + ch * _B, _B)
        pltpu.sync_copy(src.at[pl.ds(e0, _B)], sidx.at[b])
        pltpu.sync_copy(dst.at[pl.ds(e0, _B)], didx.at[b])

    stage(0, 0)
    pltpu.async_copy(table.at[sidx.at[0]], rows.at[0], sems[0])

    def pair(i, carry):
        for b in (0, 1):
            ch = i * 2 + b
            nb = 1 - b

            @pl.when(ch + 1 < _NCH)
            def _():
                stage(ch + 1, nb)
                pltpu.async_copy(table.at[sidx.at[nb]], rows.at[nb], sems[nb])

            pltpu.make_async_copy(table.at[sidx.at[b]], rows.at[b],
                                  sems[b]).wait()
            pltpu.sync_copy(rows.at[b], acc.at[didx.at[b]], add=True)
        return carry

    lax.fori_loop(0, _NCH // 2, pair, 0)


def _count_dir(s, dst, sidx, didx, ones_v, acc):
    def chunk(i, carry):
        e0 = pl.multiple_of(s * _ET + i * _B, _B)
        pltpu.sync_copy(dst.at[pl.ds(e0, _B)], didx.at[0])
        pltpu.sync_copy(ones_v, acc.at[didx.at[0]], add=True)
        return carry
    lax.fori_loop(0, _NCH, chunk, 0)


def _zero_rows(zfeat, acc, r0):
    pltpu.sync_copy(zfeat.at[pl.ds(r0, _RT), :], acc.at[pl.ds(r0, _RT), :])


def _writeout(acc, out, r0):
    pltpu.sync_copy(acc.at[pl.ds(r0, _RT), :], out.at[pl.ds(r0, _RT), :])


def _agg1_body(hu0, hu1, hw0, hw1, s_uw, d_uw, s_wu, d_wu, zfeat, ones_h,
               sw0, sw1, su0, su1, cw, cu,
               acc, rows, sidx, didx, ones_v, sem0, sem1):
    c = lax.axis_index("c")
    s = lax.axis_index("s")
    r0 = s * _RT
    sems = (sem0, sem1)

    pltpu.sync_copy(ones_h, ones_v)
    _zero_rows(zfeat, acc, r0)
    plsc.subcore_barrier()

    @pl.when(c == 0)
    def _():
        _count_dir(s, d_uw, sidx, didx, ones_v, acc)

    @pl.when(c == 1)
    def _():
        _count_dir(s, d_wu, sidx, didx, ones_v, acc)

    plsc.subcore_barrier()

    @pl.when(c == 0)
    def _():
        _writeout(acc, cw, r0)

    @pl.when(c == 1)
    def _():
        _writeout(acc, cu, r0)

    _zero_rows(zfeat, acc, r0)
    plsc.subcore_barrier()

    @pl.when(c == 0)
    def _():
        _gather_scatter_dir(s, hu0, s_uw, d_uw, sidx, didx, rows, acc, sems)

    @pl.when(c == 1)
    def _():
        _gather_scatter_dir(s, hu1, s_uw, d_uw, sidx, didx, rows, acc, sems)

    plsc.subcore_barrier()

    @pl.when(c == 0)
    def _():
        _writeout(acc, sw0, r0)

    @pl.when(c == 1)
    def _():
        _writeout(acc, sw1, r0)

    _zero_rows(zfeat, acc, r0)
    plsc.subcore_barrier()

    @pl.when(c == 0)
    def _():
        _gather_scatter_dir(s, hw0, s_wu, d_wu, sidx, didx, rows, acc, sems)

    @pl.when(c == 1)
    def _():
        _gather_scatter_dir(s, hw1, s_wu, d_wu, sidx, didx, rows, acc, sems)

    plsc.subcore_barrier()

    @pl.when(c == 0)
    def _():
        _writeout(acc, su0, r0)

    @pl.when(c == 1)
    def _():
        _writeout(acc, su1, r0)


def _make_agg1():
    mesh = plsc.VectorSubcoreMesh(core_axis_name="c", subcore_axis_name="s")
    out_type = [jax.ShapeDtypeStruct((_NP, 128), jnp.float32)] * 6
    scratch = [
        pltpu.VMEM_SHARED((_NP, 128), jnp.float32),
        pltpu.VMEM((2, _B, 128), jnp.float32),
        pltpu.VMEM((2, _B), jnp.int32),
        pltpu.VMEM((2, _B), jnp.int32),
        pltpu.VMEM((_B, 128), jnp.float32),
        pltpu.SemaphoreType.DMA,
        pltpu.SemaphoreType.DMA,
    ]
    return pl.kernel(
        _agg1_body,
        out_type=out_type,
        mesh=mesh,
        scratch_types=scratch,
    )


def _agg2_body(pu, pw, s_uw, d_uw, s_wu, d_wu, zfeat,
               s2w, s2u,
               acc, rows, sidx, didx, sem0, sem1):
    c = lax.axis_index("c")
    s = lax.axis_index("s")
    r0 = s * _RT
    sems = (sem0, sem1)

    _zero_rows(zfeat, acc, r0)
    plsc.subcore_barrier()

    @pl.when(c == 0)
    def _():
        _gather_scatter_dir(s, pu, s_uw, d_uw, sidx, didx, rows, acc, sems)

    @pl.when(c == 1)
    def _():
        _gather_scatter_dir(s, pw, s_wu, d_wu, sidx, didx, rows, acc, sems)

    plsc.subcore_barrier()

    @pl.when(c == 0)
    def _():
        _writeout(acc, s2w, r0)

    @pl.when(c == 1)
    def _():
        _writeout(acc, s2u, r0)


def _make_agg2():
    mesh = plsc.VectorSubcoreMesh(core_axis_name="c", subcore_axis_name="s")
    out_type = [jax.ShapeDtypeStruct((_NP, 128), jnp.float32)] * 2
    scratch = [
        pltpu.VMEM_SHARED((_NP, 128), jnp.float32),
        pltpu.VMEM((2, _B, 128), jnp.float32),
        pltpu.VMEM((2, _B), jnp.int32),
        pltpu.VMEM((2, _B), jnp.int32),
        pltpu.SemaphoreType.DMA,
        pltpu.SemaphoreType.DMA,
    ]
    return pl.kernel(
        _agg2_body,
        out_type=out_type,
        mesh=mesh,
        scratch_types=scratch,
    )


def _comb_body(s0_ref, s1_ref, cnt_ref, h0_ref, h1_ref, l_ref, r_ref, b_ref,
               c2_ref, o_ref, p_ref):
    inv = 1.0 / jnp.maximum(cnt_ref[...][:, :1], 1.0)
    lw = l_ref[...]
    rw = r_ref[...]
    o = (_dot(s0_ref[...] * inv, lw[:128, :])
         + _dot(s1_ref[...] * inv, lw[128:, :])
         + _dot(h0_ref[...], rw[:128, :])
         + _dot(h1_ref[...], rw[128:, :])
         + b_ref[...])
    o = jnp.maximum(o, 0.0)
    o_ref[...] = o
    p = _dot(o, c2_ref[...])
    p_ref[...] = jnp.concatenate(
        [p, jnp.zeros((p.shape[0], 128 - _OUT), jnp.float32)], axis=1)


def _comb(s0, s1, cnt, h0, h1, lw, rw, b2, c2):
    return pl.pallas_call(
        _comb_body,
        grid=(_N // _BN,),
        in_specs=[
            pl.BlockSpec((_BN, 128), lambda i: (i, 0)),
            pl.BlockSpec((_BN, 128), lambda i: (i, 0)),
            pl.BlockSpec((_BN, 128), lambda i: (i, 0)),
            pl.BlockSpec((_BN, 128), lambda i: (i, 0)),
            pl.BlockSpec((_BN, 128), lambda i: (i, 0)),
            pl.BlockSpec((_HID, _HID), lambda i: (0, 0)),
            pl.BlockSpec((_HID, _HID), lambda i: (0, 0)),
            pl.BlockSpec((1, _HID), lambda i: (0, 0)),
            pl.BlockSpec((_HID, _OUT), lambda i: (0, 0)),
        ],
        out_specs=[
            pl.BlockSpec((_BN, _HID), lambda i: (i, 0)),
            pl.BlockSpec((_BN, 128), lambda i: (i, 0)),
        ],
        out_shape=[
            jax.ShapeDtypeStruct((_N, _HID), jnp.float32),
            jax.ShapeDtypeStruct((_N, 128), jnp.float32),
        ],
    )(s0, s1, cnt, h0, h1, lw, rw, b2, c2)


def _final_body(t_ref, cnt_ref, o_ref, r_ref, b_ref, z_ref):
    inv = 1.0 / jnp.maximum(cnt_ref[...][:, :1], 1.0)
    s2 = t_ref[...][:, :_OUT]
    z = s2 * inv + _dot(o_ref[...], r_ref[...]) + b_ref[...]
    m = jnp.max(z, axis=1, keepdims=True)
    ez = jnp.exp(z - m)
    lse = jnp.log(jnp.sum(ez, axis=1, keepdims=True))
    z_ref[...] = z - m - lse


def _final(t, cnt, o, rw, b2):
    return pl.pallas_call(
        _final_body,
        grid=(_N // _BN,),
        in_specs=[
            pl.BlockSpec((_BN, 128), lambda i: (i, 0)),
            pl.BlockSpec((_BN, 128), lambda i: (i, 0)),
            pl.BlockSpec((_BN, _HID), lambda i: (i, 0)),
            pl.BlockSpec((_HID, _OUT), lambda i: (0, 0)),
            pl.BlockSpec((1, _OUT), lambda i: (0, 0)),
        ],
        out_specs=pl.BlockSpec((_BN, _OUT), lambda i: (i, 0)),
        out_shape=jax.ShapeDtypeStruct((_N, _OUT), jnp.float32),
    )(t, cnt, o, rw, b2)


def kernel(x_websites, x_users, ei_u2w, ei_w2u,
           lin_w_web, lin_b_web, lin_w_usr, lin_b_usr,
           c1_uw_l, c1_uw_r, c1_uw_b, c1_wu_l, c1_wu_r, c1_wu_b,
           c2_uw_l, c2_uw_r, c2_uw_b, c2_wu_l, c2_wu_r, c2_wu_b):
    z128 = jnp.zeros((_NP, 128), jnp.float32)
    ones_h = jnp.ones((_B, 128), jnp.float32)
    s_uw, d_uw = ei_u2w[0], ei_u2w[1]
    s_wu, d_wu = ei_w2u[0], ei_w2u[1]

    h_w0, h_w1 = _proj(x_websites, lin_w_web, lin_b_web.reshape(1, -1))
    h_u0, h_u1 = _proj(x_users, lin_w_usr, lin_b_usr.reshape(1, -1))

    sw0, sw1, su0, su1, cw, cu = _make_agg1()(
        h_u0, h_u1, h_w0, h_w1, s_uw, d_uw, s_wu, d_wu, z128, ones_h)

    o_w, p_w = _comb(sw0, sw1, cw, h_w0, h_w1,
                     c1_uw_l, c1_uw_r, c1_uw_b.reshape(1, -1), c2_wu_l)
    o_u, p_u = _comb(su0, su1, cu, h_u0, h_u1,
                     c1_wu_l, c1_wu_r, c1_wu_b.reshape(1, -1), c2_uw_l)

    s2w, s2u = _make_agg2()(
        p_u, p_w, s_uw, d_uw, s_wu, d_wu, z128)

    z_w = _final(s2w, cw, o_w, c2_uw_r, c2_uw_b.reshape(1, -1))
    z_u = _final(s2u, cu, o_u, c2_wu_r, c2_wu_b.reshape(1, -1))
    return (z_w, z_u)


# speedup vs baseline: 7.3540x; 1.4207x over previous
import functools

import jax
import jax.numpy as jnp
from jax import lax
from jax.experimental import pallas as pl
from jax.experimental.pallas import tpu as pltpu
from jax.experimental.pallas import tpu_sc as plsc

_N = 10000
_D_IN = 128
_HID = 256
_OUT = 64
_E = 320000

_NT = 16
_NP = 10240
_RT = _NP // _NT
_B = 80
_ET = _E // _NT
_NCH = _ET // _B
_IB = 50
_NB = _NCH // _IB

_BN = 1000


def _dot(a, b):
    return jnp.dot(a, b, precision=lax.Precision.HIGHEST,
                   preferred_element_type=jnp.float32)


def _proj_body(x_ref, w_ref, b_ref, h0_ref, h1_ref):
    h = _dot(x_ref[...], w_ref[...]) + b_ref[...]
    h = jnp.maximum(h, 0.0)
    h0_ref[...] = h[:, :128]
    h1_ref[...] = h[:, 128:]


def _proj(x, w, b2):
    return pl.pallas_call(
        _proj_body,
        grid=(_N // _BN,),
        in_specs=[
            pl.BlockSpec((_BN, _D_IN), lambda i: (i, 0)),
            pl.BlockSpec((_D_IN, _HID), lambda i: (0, 0)),
            pl.BlockSpec((1, _HID), lambda i: (0, 0)),
        ],
        out_specs=[
            pl.BlockSpec((_BN, 128), lambda i: (i, 0)),
            pl.BlockSpec((_BN, 128), lambda i: (i, 0)),
        ],
        out_shape=[jax.ShapeDtypeStruct((_N, 128), jnp.float32)] * 2,
    )(x, w, b2)


def _gather_scatter_dir(s, table, src3, dst3, sidx, didx, rows, acc, sems):
    def blk(k, carry):
        pltpu.sync_copy(src3.at[s * _NB + k], sidx)
        pltpu.sync_copy(dst3.at[s * _NB + k], didx)
        pltpu.async_copy(table.at[sidx.at[0]], rows.at[0], sems[0])

        def pair(i, c2):
            for b in (0, 1):
                ch = i * 2 + b
                nb = 1 - b

                @pl.when(ch + 1 < _IB)
                def _():
                    pltpu.async_copy(table.at[sidx.at[ch + 1]], rows.at[nb],
                                     sems[nb])

                pltpu.make_async_copy(table.at[sidx.at[ch]], rows.at[b],
                                      sems[b]).wait()
                pltpu.sync_copy(rows.at[b], acc.at[didx.at[ch]], add=True)
            return c2

        lax.fori_loop(0, _IB // 2, pair, 0)
        return carry

    lax.fori_loop(0, _NB, blk, 0)


_CB = 25


def _count_dir(s, dst3, didx, ones_v, acc, sem):
    def blk(k, carry):
        pltpu.sync_copy(dst3.at[s * _NB + k], didx)

        def burst(i, c2):
            for j in range(_CB):
                pltpu.async_copy(ones_v, acc.at[didx.at[i * _CB + j]], sem,
                                 add=True)
            for j in range(_CB):
                pltpu.make_async_copy(ones_v, acc.at[didx.at[i * _CB + j]],
                                      sem).wait()
            return c2

        lax.fori_loop(0, _IB // _CB, burst, 0)
        return carry

    lax.fori_loop(0, _NB, blk, 0)


def _zero_rows(zfeat, acc, r0):
    pltpu.sync_copy(zfeat.at[pl.ds(r0, _RT), :], acc.at[pl.ds(r0, _RT), :])


def _writeout(acc, out, r0):
    pltpu.sync_copy(acc.at[pl.ds(r0, _RT), :], out.at[pl.ds(r0, _RT), :])


def _agg1_body(hu0, hu1, hw0, hw1, s_uw, d_uw, s_wu, d_wu, zfeat, ones_h,
               sw0, sw1, su0, su1, cw, cu,
               acc, rows, sidx, didx, ones_v, sem0, sem1):
    c = lax.axis_index("c")
    s = lax.axis_index("s")
    r0 = s * _RT
    sems = (sem0, sem1)

    pltpu.sync_copy(ones_h, ones_v)
    _zero_rows(zfeat, acc, r0)
    plsc.subcore_barrier()

    @pl.when(c == 0)
    def _():
        _count_dir(s, d_uw, didx, ones_v, acc, sem0)

    @pl.when(c == 1)
    def _():
        _count_dir(s, d_wu, didx, ones_v, acc, sem0)

    plsc.subcore_barrier()

    @pl.when(c == 0)
    def _():
        _writeout(acc, cw, r0)

    @pl.when(c == 1)
    def _():
        _writeout(acc, cu, r0)

    _zero_rows(zfeat, acc, r0)
    plsc.subcore_barrier()

    @pl.when(c == 0)
    def _():
        _gather_scatter_dir(s, hu0, s_uw, d_uw, sidx, didx, rows, acc, sems)

    @pl.when(c == 1)
    def _():
        _gather_scatter_dir(s, hu1, s_uw, d_uw, sidx, didx, rows, acc, sems)

    plsc.subcore_barrier()

    @pl.when(c == 0)
    def _():
        _writeout(acc, sw0, r0)

    @pl.when(c == 1)
    def _():
        _writeout(acc, sw1, r0)

    _zero_rows(zfeat, acc, r0)
    plsc.subcore_barrier()

    @pl.when(c == 0)
    def _():
        _gather_scatter_dir(s, hw0, s_wu, d_wu, sidx, didx, rows, acc, sems)

    @pl.when(c == 1)
    def _():
        _gather_scatter_dir(s, hw1, s_wu, d_wu, sidx, didx, rows, acc, sems)

    plsc.subcore_barrier()

    @pl.when(c == 0)
    def _():
        _writeout(acc, su0, r0)

    @pl.when(c == 1)
    def _():
        _writeout(acc, su1, r0)


def _make_agg1():
    mesh = plsc.VectorSubcoreMesh(core_axis_name="c", subcore_axis_name="s")
    out_type = [jax.ShapeDtypeStruct((_NP, 128), jnp.float32)] * 6
    scratch = [
        pltpu.VMEM_SHARED((_NP, 128), jnp.float32),
        pltpu.VMEM((2, _B, 128), jnp.float32),
        pltpu.VMEM((_IB, _B), jnp.int32),
        pltpu.VMEM((_IB, _B), jnp.int32),
        pltpu.VMEM((_B, 128), jnp.float32),
        pltpu.SemaphoreType.DMA,
        pltpu.SemaphoreType.DMA,
    ]
    return pl.kernel(
        _agg1_body,
        out_type=out_type,
        mesh=mesh,
        scratch_types=scratch,
    )


def _agg2_body(pu, pw, s_uw, d_uw, s_wu, d_wu, zfeat,
               s2w, s2u,
               acc, rows, sidx, didx, sem0, sem1):
    c = lax.axis_index("c")
    s = lax.axis_index("s")
    r0 = s * _RT
    sems = (sem0, sem1)

    _zero_rows(zfeat, acc, r0)
    plsc.subcore_barrier()

    @pl.when(c == 0)
    def _():
        _gather_scatter_dir(s, pu, s_uw, d_uw, sidx, didx, rows, acc, sems)

    @pl.when(c == 1)
    def _():
        _gather_scatter_dir(s, pw, s_wu, d_wu, sidx, didx, rows, acc, sems)

    plsc.subcore_barrier()

    @pl.when(c == 0)
    def _():
        _writeout(acc, s2w, r0)

    @pl.when(c == 1)
    def _():
        _writeout(acc, s2u, r0)


def _make_agg2():
    mesh = plsc.VectorSubcoreMesh(core_axis_name="c", subcore_axis_name="s")
    out_type = [jax.ShapeDtypeStruct((_NP, 128), jnp.float32)] * 2
    scratch = [
        pltpu.VMEM_SHARED((_NP, 128), jnp.float32),
        pltpu.VMEM((2, _B, 128), jnp.float32),
        pltpu.VMEM((_IB, _B), jnp.int32),
        pltpu.VMEM((_IB, _B), jnp.int32),
        pltpu.SemaphoreType.DMA,
        pltpu.SemaphoreType.DMA,
    ]
    return pl.kernel(
        _agg2_body,
        out_type=out_type,
        mesh=mesh,
        scratch_types=scratch,
    )


def _comb_body(s0_ref, s1_ref, cnt_ref, h0_ref, h1_ref, l_ref, r_ref, b_ref,
               c2_ref, o_ref, p_ref):
    inv = 1.0 / jnp.maximum(cnt_ref[...][:, :1], 1.0)
    lw = l_ref[...]
    rw = r_ref[...]
    o = (_dot(s0_ref[...] * inv, lw[:128, :])
         + _dot(s1_ref[...] * inv, lw[128:, :])
         + _dot(h0_ref[...], rw[:128, :])
         + _dot(h1_ref[...], rw[128:, :])
         + b_ref[...])
    o = jnp.maximum(o, 0.0)
    o_ref[...] = o
    p = _dot(o, c2_ref[...])
    p_ref[...] = jnp.concatenate(
        [p, jnp.zeros((p.shape[0], 128 - _OUT), jnp.float32)], axis=1)


def _comb(s0, s1, cnt, h0, h1, lw, rw, b2, c2):
    return pl.pallas_call(
        _comb_body,
        grid=(_N // _BN,),
        in_specs=[
            pl.BlockSpec((_BN, 128), lambda i: (i, 0)),
            pl.BlockSpec((_BN, 128), lambda i: (i, 0)),
            pl.BlockSpec((_BN, 128), lambda i: (i, 0)),
            pl.BlockSpec((_BN, 128), lambda i: (i, 0)),
            pl.BlockSpec((_BN, 128), lambda i: (i, 0)),
            pl.BlockSpec((_HID, _HID), lambda i: (0, 0)),
            pl.BlockSpec((_HID, _HID), lambda i: (0, 0)),
            pl.BlockSpec((1, _HID), lambda i: (0, 0)),
            pl.BlockSpec((_HID, _OUT), lambda i: (0, 0)),
        ],
        out_specs=[
            pl.BlockSpec((_BN, _HID), lambda i: (i, 0)),
            pl.BlockSpec((_BN, 128), lambda i: (i, 0)),
        ],
        out_shape=[
            jax.ShapeDtypeStruct((_N, _HID), jnp.float32),
            jax.ShapeDtypeStruct((_N, 128), jnp.float32),
        ],
    )(s0, s1, cnt, h0, h1, lw, rw, b2, c2)


def _final_body(t_ref, cnt_ref, o_ref, r_ref, b_ref, z_ref):
    inv = 1.0 / jnp.maximum(cnt_ref[...][:, :1], 1.0)
    s2 = t_ref[...][:, :_OUT]
    z = s2 * inv + _dot(o_ref[...], r_ref[...]) + b_ref[...]
    m = jnp.max(z, axis=1, keepdims=True)
    ez = jnp.exp(z - m)
    lse = jnp.log(jnp.sum(ez, axis=1, keepdims=True))
    z_ref[...] = z - m - lse


def _final(t, cnt, o, rw, b2):
    return pl.pallas_call(
        _final_body,
        grid=(_N // _BN,),
        in_specs=[
            pl.BlockSpec((_BN, 128), lambda i: (i, 0)),
            pl.BlockSpec((_BN, 128), lambda i: (i, 0)),
            pl.BlockSpec((_BN, _HID), lambda i: (i, 0)),
            pl.BlockSpec((_HID, _OUT), lambda i: (0, 0)),
            pl.BlockSpec((1, _OUT), lambda i: (0, 0)),
        ],
        out_specs=pl.BlockSpec((_BN, _OUT), lambda i: (i, 0)),
        out_shape=jax.ShapeDtypeStruct((_N, _OUT), jnp.float32),
    )(t, cnt, o, rw, b2)


def kernel(x_websites, x_users, ei_u2w, ei_w2u,
           lin_w_web, lin_b_web, lin_w_usr, lin_b_usr,
           c1_uw_l, c1_uw_r, c1_uw_b, c1_wu_l, c1_wu_r, c1_wu_b,
           c2_uw_l, c2_uw_r, c2_uw_b, c2_wu_l, c2_wu_r, c2_wu_b):
    z128 = jnp.zeros((_NP, 128), jnp.float32)
    ones_h = jnp.ones((_B, 128), jnp.float32)
    s_uw = ei_u2w[0].reshape(_NT * _NB, _IB, _B)
    d_uw = ei_u2w[1].reshape(_NT * _NB, _IB, _B)
    s_wu = ei_w2u[0].reshape(_NT * _NB, _IB, _B)
    d_wu = ei_w2u[1].reshape(_NT * _NB, _IB, _B)

    h_w0, h_w1 = _proj(x_websites, lin_w_web, lin_b_web.reshape(1, -1))
    h_u0, h_u1 = _proj(x_users, lin_w_usr, lin_b_usr.reshape(1, -1))

    sw0, sw1, su0, su1, cw, cu = _make_agg1()(
        h_u0, h_u1, h_w0, h_w1, s_uw, d_uw, s_wu, d_wu, z128, ones_h)

    o_w, p_w = _comb(sw0, sw1, cw, h_w0, h_w1,
                     c1_uw_l, c1_uw_r, c1_uw_b.reshape(1, -1), c2_wu_l)
    o_u, p_u = _comb(su0, su1, cu, h_u0, h_u1,
                     c1_wu_l, c1_wu_r, c1_wu_b.reshape(1, -1), c2_uw_l)

    s2w, s2u = _make_agg2()(
        p_u, p_w, s_uw, d_uw, s_wu, d_wu, z128)

    z_w = _final(s2w, cw, o_w, c2_uw_r, c2_uw_b.reshape(1, -1))
    z_u = _final(s2u, cu, o_u, c2_wu_r, c2_wu_b.reshape(1, -1))
    return (z_w, z_u)


# speedup vs baseline: 8.0392x; 1.0932x over previous
import functools

import jax
import jax.numpy as jnp
from jax import lax
from jax.experimental import pallas as pl
from jax.experimental.pallas import tpu as pltpu
from jax.experimental.pallas import tpu_sc as plsc

_N = 10000
_D_IN = 128
_HID = 256
_OUT = 64
_E = 320000

_NT = 16
_NP = 10240
_RT = _NP // _NT
_B = 80
_ET = _E // _NT
_NCH = _ET // _B
_IB = 50
_NB = _NCH // _IB

_BN = 1000


def _dot(a, b):
    return jnp.dot(a, b, preferred_element_type=jnp.float32)


def _proj_body(x_ref, w_ref, b_ref, h0_ref, h1_ref):
    h = _dot(x_ref[...], w_ref[...]) + b_ref[...]
    h = jnp.maximum(h, 0.0)
    h0_ref[...] = h[:, :128]
    h1_ref[...] = h[:, 128:]


def _proj(x, w, b2):
    return pl.pallas_call(
        _proj_body,
        grid=(_N // _BN,),
        in_specs=[
            pl.BlockSpec((_BN, _D_IN), lambda i: (i, 0)),
            pl.BlockSpec((_D_IN, _HID), lambda i: (0, 0)),
            pl.BlockSpec((1, _HID), lambda i: (0, 0)),
        ],
        out_specs=[
            pl.BlockSpec((_BN, 128), lambda i: (i, 0)),
            pl.BlockSpec((_BN, 128), lambda i: (i, 0)),
        ],
        out_shape=[jax.ShapeDtypeStruct((_N, 128), jnp.float32)] * 2,
    )(x, w, b2)


def _gather_scatter_dir(s, table, src3, dst3, sidx, didx, rows, acc, sems):
    def blk(k, carry):
        pltpu.sync_copy(src3.at[s * _NB + k], sidx)
        pltpu.sync_copy(dst3.at[s * _NB + k], didx)
        pltpu.async_copy(table.at[sidx.at[0]], rows.at[0], sems[0])

        def pair(i, c2):
            for b in (0, 1):
                ch = i * 2 + b
                nb = 1 - b

                @pl.when(ch + 1 < _IB)
                def _():
                    pltpu.async_copy(table.at[sidx.at[ch + 1]], rows.at[nb],
                                     sems[nb])

                pltpu.make_async_copy(table.at[sidx.at[ch]], rows.at[b],
                                      sems[b]).wait()
                pltpu.sync_copy(rows.at[b], acc.at[didx.at[ch]], add=True)
            return c2

        lax.fori_loop(0, _IB // 2, pair, 0)
        return carry

    lax.fori_loop(0, _NB, blk, 0)


_CB = 25


def _count_dir(s, dst3, didx, ones_v, acc, sem):
    def blk(k, carry):
        pltpu.sync_copy(dst3.at[s * _NB + k], didx)

        def burst(i, c2):
            for j in range(_CB):
                pltpu.async_copy(ones_v, acc.at[didx.at[i * _CB + j]], sem,
                                 add=True)
            for j in range(_CB):
                pltpu.make_async_copy(ones_v, acc.at[didx.at[i * _CB + j]],
                                      sem).wait()
            return c2

        lax.fori_loop(0, _IB // _CB, burst, 0)
        return carry

    lax.fori_loop(0, _NB, blk, 0)


def _zero_rows(zfeat, acc, r0):
    pltpu.sync_copy(zfeat.at[pl.ds(r0, _RT), :], acc.at[pl.ds(r0, _RT), :])


def _writeout(acc, out, r0):
    pltpu.sync_copy(acc.at[pl.ds(r0, _RT), :], out.at[pl.ds(r0, _RT), :])


def _agg1_body(hu0, hu1, hw0, hw1, s_uw, d_uw, s_wu, d_wu, zfeat,
               sw0, sw1, su0, su1,
               acc, rows, sidx, didx, sem0, sem1):
    c = lax.axis_index("c")
    s = lax.axis_index("s")
    r0 = s * _RT
    sems = (sem0, sem1)

    _zero_rows(zfeat, acc, r0)
    plsc.subcore_barrier()

    @pl.when(c == 0)
    def _():
        _gather_scatter_dir(s, hu0, s_uw, d_uw, sidx, didx, rows, acc, sems)

    @pl.when(c == 1)
    def _():
        _gather_scatter_dir(s, hu1, s_uw, d_uw, sidx, didx, rows, acc, sems)

    plsc.subcore_barrier()

    @pl.when(c == 0)
    def _():
        _writeout(acc, sw0, r0)

    @pl.when(c == 1)
    def _():
        _writeout(acc, sw1, r0)

    _zero_rows(zfeat, acc, r0)
    plsc.subcore_barrier()

    @pl.when(c == 0)
    def _():
        _gather_scatter_dir(s, hw0, s_wu, d_wu, sidx, didx, rows, acc, sems)

    @pl.when(c == 1)
    def _():
        _gather_scatter_dir(s, hw1, s_wu, d_wu, sidx, didx, rows, acc, sems)

    plsc.subcore_barrier()

    @pl.when(c == 0)
    def _():
        _writeout(acc, su0, r0)

    @pl.when(c == 1)
    def _():
        _writeout(acc, su1, r0)


def _make_agg1():
    mesh = plsc.VectorSubcoreMesh(core_axis_name="c", subcore_axis_name="s")
    out_type = [jax.ShapeDtypeStruct((_NP, 128), jnp.float32)] * 4
    scratch = [
        pltpu.VMEM_SHARED((_NP, 128), jnp.float32),
        pltpu.VMEM((2, _B, 128), jnp.float32),
        pltpu.VMEM((_IB, _B), jnp.int32),
        pltpu.VMEM((_IB, _B), jnp.int32),
        pltpu.SemaphoreType.DMA,
        pltpu.SemaphoreType.DMA,
    ]
    return pl.kernel(
        _agg1_body,
        out_type=out_type,
        mesh=mesh,
        scratch_types=scratch,
    )


def _cnt_body(d_uw, d_wu, zfeat, ones_h,
              cw, cu,
              acc, didx, ones_v, sem0):
    c = lax.axis_index("c")
    s = lax.axis_index("s")
    r0 = s * _RT

    pltpu.sync_copy(ones_h, ones_v)
    _zero_rows(zfeat, acc, r0)
    plsc.subcore_barrier()

    @pl.when(c == 0)
    def _():
        _count_dir(s, d_uw, didx, ones_v, acc, sem0)

    @pl.when(c == 1)
    def _():
        _count_dir(s, d_wu, didx, ones_v, acc, sem0)

    plsc.subcore_barrier()

    @pl.when(c == 0)
    def _():
        _writeout(acc, cw, r0)

    @pl.when(c == 1)
    def _():
        _writeout(acc, cu, r0)


def _make_cnt():
    mesh = plsc.VectorSubcoreMesh(core_axis_name="c", subcore_axis_name="s")
    out_type = [jax.ShapeDtypeStruct((_NP, 128), jnp.float32)] * 2
    scratch = [
        pltpu.VMEM_SHARED((_NP, 128), jnp.float32),
        pltpu.VMEM((_IB, _B), jnp.int32),
        pltpu.VMEM((_B, 128), jnp.float32),
        pltpu.SemaphoreType.DMA,
    ]
    return pl.kernel(
        _cnt_body,
        out_type=out_type,
        mesh=mesh,
        scratch_types=scratch,
    )


def _agg2_body(pu, pw, s_uw, d_uw, s_wu, d_wu, zfeat,
               s2w, s2u,
               acc, rows, sidx, didx, sem0, sem1):
    c = lax.axis_index("c")
    s = lax.axis_index("s")
    r0 = s * _RT
    sems = (sem0, sem1)

    _zero_rows(zfeat, acc, r0)
    plsc.subcore_barrier()

    @pl.when(c == 0)
    def _():
        _gather_scatter_dir(s, pu, s_uw, d_uw, sidx, didx, rows, acc, sems)

    @pl.when(c == 1)
    def _():
        _gather_scatter_dir(s, pw, s_wu, d_wu, sidx, didx, rows, acc, sems)

    plsc.subcore_barrier()

    @pl.when(c == 0)
    def _():
        _writeout(acc, s2w, r0)

    @pl.when(c == 1)
    def _():
        _writeout(acc, s2u, r0)


def _make_agg2():
    mesh = plsc.VectorSubcoreMesh(core_axis_name="c", subcore_axis_name="s")
    out_type = [jax.ShapeDtypeStruct((_NP, 128), jnp.float32)] * 2
    scratch = [
        pltpu.VMEM_SHARED((_NP, 128), jnp.float32),
        pltpu.VMEM((2, _B, 128), jnp.float32),
        pltpu.VMEM((_IB, _B), jnp.int32),
        pltpu.VMEM((_IB, _B), jnp.int32),
        pltpu.SemaphoreType.DMA,
        pltpu.SemaphoreType.DMA,
    ]
    return pl.kernel(
        _agg2_body,
        out_type=out_type,
        mesh=mesh,
        scratch_types=scratch,
    )


def _comb_body(s0_ref, s1_ref, cnt_ref, h0_ref, h1_ref, l_ref, r_ref, b_ref,
               c2_ref, o_ref, p_ref):
    inv = 1.0 / jnp.maximum(cnt_ref[...][:, :1], 1.0)
    lw = l_ref[...]
    rw = r_ref[...]
    o = (_dot(s0_ref[...] * inv, lw[:128, :])
         + _dot(s1_ref[...] * inv, lw[128:, :])
         + _dot(h0_ref[...], rw[:128, :])
         + _dot(h1_ref[...], rw[128:, :])
         + b_ref[...])
    o = jnp.maximum(o, 0.0)
    o_ref[...] = o
    p = _dot(o, c2_ref[...])
    p_ref[...] = jnp.concatenate(
        [p, jnp.zeros((p.shape[0], 128 - _OUT), jnp.float32)], axis=1)


def _comb(s0, s1, cnt, h0, h1, lw, rw, b2, c2):
    return pl.pallas_call(
        _comb_body,
        grid=(_N // _BN,),
        in_specs=[
            pl.BlockSpec((_BN, 128), lambda i: (i, 0)),
            pl.BlockSpec((_BN, 128), lambda i: (i, 0)),
            pl.BlockSpec((_BN, 128), lambda i: (i, 0)),
            pl.BlockSpec((_BN, 128), lambda i: (i, 0)),
            pl.BlockSpec((_BN, 128), lambda i: (i, 0)),
            pl.BlockSpec((_HID, _HID), lambda i: (0, 0)),
            pl.BlockSpec((_HID, _HID), lambda i: (0, 0)),
            pl.BlockSpec((1, _HID), lambda i: (0, 0)),
            pl.BlockSpec((_HID, _OUT), lambda i: (0, 0)),
        ],
        out_specs=[
            pl.BlockSpec((_BN, _HID), lambda i: (i, 0)),
            pl.BlockSpec((_BN, 128), lambda i: (i, 0)),
        ],
        out_shape=[
            jax.ShapeDtypeStruct((_N, _HID), jnp.float32),
            jax.ShapeDtypeStruct((_N, 128), jnp.float32),
        ],
    )(s0, s1, cnt, h0, h1, lw, rw, b2, c2)


def _final_body(t_ref, cnt_ref, o_ref, r_ref, b_ref, z_ref):
    inv = 1.0 / jnp.maximum(cnt_ref[...][:, :1], 1.0)
    s2 = t_ref[...][:, :_OUT]
    z = s2 * inv + _dot(o_ref[...], r_ref[...]) + b_ref[...]
    m = jnp.max(z, axis=1, keepdims=True)
    ez = jnp.exp(z - m)
    lse = jnp.log(jnp.sum(ez, axis=1, keepdims=True))
    z_ref[...] = z - m - lse


def _final(t, cnt, o, rw, b2):
    return pl.pallas_call(
        _final_body,
        grid=(_N // _BN,),
        in_specs=[
            pl.BlockSpec((_BN, 128), lambda i: (i, 0)),
            pl.BlockSpec((_BN, 128), lambda i: (i, 0)),
            pl.BlockSpec((_BN, _HID), lambda i: (i, 0)),
            pl.BlockSpec((_HID, _OUT), lambda i: (0, 0)),
            pl.BlockSpec((1, _OUT), lambda i: (0, 0)),
        ],
        out_specs=pl.BlockSpec((_BN, _OUT), lambda i: (i, 0)),
        out_shape=jax.ShapeDtypeStruct((_N, _OUT), jnp.float32),
    )(t, cnt, o, rw, b2)


def kernel(x_websites, x_users, ei_u2w, ei_w2u,
           lin_w_web, lin_b_web, lin_w_usr, lin_b_usr,
           c1_uw_l, c1_uw_r, c1_uw_b, c1_wu_l, c1_wu_r, c1_wu_b,
           c2_uw_l, c2_uw_r, c2_uw_b, c2_wu_l, c2_wu_r, c2_wu_b):
    z128 = jnp.zeros((_NP, 128), jnp.float32)
    ones_h = jnp.ones((_B, 128), jnp.float32)
    s_uw = ei_u2w[0].reshape(_NT * _NB, _IB, _B)
    d_uw = ei_u2w[1].reshape(_NT * _NB, _IB, _B)
    s_wu = ei_w2u[0].reshape(_NT * _NB, _IB, _B)
    d_wu = ei_w2u[1].reshape(_NT * _NB, _IB, _B)

    cw, cu = _make_cnt()(d_uw, d_wu, z128, ones_h)

    h_w0, h_w1 = _proj(x_websites, lin_w_web, lin_b_web.reshape(1, -1))
    h_u0, h_u1 = _proj(x_users, lin_w_usr, lin_b_usr.reshape(1, -1))

    sw0, sw1, su0, su1 = _make_agg1()(
        h_u0, h_u1, h_w0, h_w1, s_uw, d_uw, s_wu, d_wu, z128)

    o_w, p_w = _comb(sw0, sw1, cw, h_w0, h_w1,
                     c1_uw_l, c1_uw_r, c1_uw_b.reshape(1, -1), c2_wu_l)
    o_u, p_u = _comb(su0, su1, cu, h_u0, h_u1,
                     c1_wu_l, c1_wu_r, c1_wu_b.reshape(1, -1), c2_uw_l)

    s2w, s2u = _make_agg2()(
        p_u, p_w, s_uw, d_uw, s_wu, d_wu, z128)

    z_w = _final(s2w, cw, o_w, c2_uw_r, c2_uw_b.reshape(1, -1))
    z_u = _final(s2u, cu, o_u, c2_wu_r, c2_wu_b.reshape(1, -1))
    return (z_w, z_u)


# speedup vs baseline: 8.6928x; 1.0813x over previous
import functools

import jax
import jax.numpy as jnp
from jax import lax
from jax.experimental import pallas as pl
from jax.experimental.pallas import tpu as pltpu
from jax.experimental.pallas import tpu_sc as plsc

_N = 10000
_D_IN = 128
_HID = 256
_OUT = 64
_E = 320000

_NT = 16
_NP = 10240
_RT = _NP // _NT
_B = 125
_ET = _E // _NT
_NCH = _ET // _B
_IB = 40
_NB = _NCH // _IB

_BN = 1000


def _dot(a, b):
    return jnp.dot(a, b, preferred_element_type=jnp.float32)


def _proj_body(x_ref, w_ref, b_ref, h0_ref, h1_ref):
    h = _dot(x_ref[...], w_ref[...]) + b_ref[...]
    h = jnp.maximum(h, 0.0)
    h0_ref[...] = h[:, :128]
    h1_ref[...] = h[:, 128:]


def _proj(x, w, b2):
    return pl.pallas_call(
        _proj_body,
        grid=(_N // _BN,),
        in_specs=[
            pl.BlockSpec((_BN, _D_IN), lambda i: (i, 0)),
            pl.BlockSpec((_D_IN, _HID), lambda i: (0, 0)),
            pl.BlockSpec((1, _HID), lambda i: (0, 0)),
        ],
        out_specs=[
            pl.BlockSpec((_BN, 128), lambda i: (i, 0)),
            pl.BlockSpec((_BN, 128), lambda i: (i, 0)),
        ],
        out_shape=[jax.ShapeDtypeStruct((_N, 128), jnp.float32)] * 2,
    )(x, w, b2)


def _gather_scatter_dir(s, table, src3, dst3, sidx, didx, rows, acc, sems):
    def blk(k, carry):
        pltpu.sync_copy(src3.at[s * _NB + k], sidx)
        pltpu.sync_copy(dst3.at[s * _NB + k], didx)
        pltpu.async_copy(table.at[sidx.at[0]], rows.at[0], sems[0])

        def pair(i, c2):
            for b in (0, 1):
                ch = i * 2 + b
                nb = 1 - b

                @pl.when(ch + 1 < _IB)
                def _():
                    pltpu.async_copy(table.at[sidx.at[ch + 1]], rows.at[nb],
                                     sems[nb])

                pltpu.make_async_copy(table.at[sidx.at[ch]], rows.at[b],
                                      sems[b]).wait()
                pltpu.sync_copy(rows.at[b], acc.at[didx.at[ch]], add=True)
            return c2

        lax.fori_loop(0, _IB // 2, pair, 0)
        return carry

    lax.fori_loop(0, _NB, blk, 0)


_CB = 40


def _count_dir(s, dst3, didx, ones_v, acc, sem):
    def blk(k, carry):
        pltpu.sync_copy(dst3.at[s * _NB + k], didx)

        def burst(i, c2):
            for j in range(_CB):
                pltpu.async_copy(ones_v, acc.at[didx.at[i * _CB + j]], sem,
                                 add=True)
            for j in range(_CB):
                pltpu.make_async_copy(ones_v, acc.at[didx.at[i * _CB + j]],
                                      sem).wait()
            return c2

        lax.fori_loop(0, _IB // _CB, burst, 0)
        return carry

    lax.fori_loop(0, _NB, blk, 0)


def _zero_rows(zfeat, acc, r0):
    pltpu.sync_copy(zfeat.at[pl.ds(r0, _RT), :], acc.at[pl.ds(r0, _RT), :])


def _writeout(acc, out, r0):
    pltpu.sync_copy(acc.at[pl.ds(r0, _RT), :], out.at[pl.ds(r0, _RT), :])


def _agg1_body(hu0, hu1, hw0, hw1, s_uw, d_uw, s_wu, d_wu, zfeat,
               sw0, sw1, su0, su1,
               acc, rows, sidx, didx, sem0, sem1):
    c = lax.axis_index("c")
    s = lax.axis_index("s")
    r0 = s * _RT
    sems = (sem0, sem1)

    _zero_rows(zfeat, acc, r0)
    plsc.subcore_barrier()

    @pl.when(c == 0)
    def _():
        _gather_scatter_dir(s, hu0, s_uw, d_uw, sidx, didx, rows, acc, sems)

    @pl.when(c == 1)
    def _():
        _gather_scatter_dir(s, hu1, s_uw, d_uw, sidx, didx, rows, acc, sems)

    plsc.subcore_barrier()

    @pl.when(c == 0)
    def _():
        _writeout(acc, sw0, r0)

    @pl.when(c == 1)
    def _():
        _writeout(acc, sw1, r0)

    _zero_rows(zfeat, acc, r0)
    plsc.subcore_barrier()

    @pl.when(c == 0)
    def _():
        _gather_scatter_dir(s, hw0, s_wu, d_wu, sidx, didx, rows, acc, sems)

    @pl.when(c == 1)
    def _():
        _gather_scatter_dir(s, hw1, s_wu, d_wu, sidx, didx, rows, acc, sems)

    plsc.subcore_barrier()

    @pl.when(c == 0)
    def _():
        _writeout(acc, su0, r0)

    @pl.when(c == 1)
    def _():
        _writeout(acc, su1, r0)


def _make_agg1():
    mesh = plsc.VectorSubcoreMesh(core_axis_name="c", subcore_axis_name="s")
    out_type = [jax.ShapeDtypeStruct((_NP, 128), jnp.float32)] * 4
    scratch = [
        pltpu.VMEM_SHARED((_NP, 128), jnp.float32),
        pltpu.VMEM((2, _B, 128), jnp.float32),
        pltpu.VMEM((_IB, _B), jnp.int32),
        pltpu.VMEM((_IB, _B), jnp.int32),
        pltpu.SemaphoreType.DMA,
        pltpu.SemaphoreType.DMA,
    ]
    return pl.kernel(
        _agg1_body,
        out_type=out_type,
        mesh=mesh,
        scratch_types=scratch,
    )


def _cnt_body(d_uw, d_wu, zfeat, ones_h,
              cw, cu,
              acc, didx, ones_v, sem0):
    c = lax.axis_index("c")
    s = lax.axis_index("s")
    r0 = s * _RT

    pltpu.sync_copy(ones_h, ones_v)
    _zero_rows(zfeat, acc, r0)
    plsc.subcore_barrier()

    @pl.when(c == 0)
    def _():
        _count_dir(s, d_uw, didx, ones_v, acc, sem0)

    @pl.when(c == 1)
    def _():
        _count_dir(s, d_wu, didx, ones_v, acc, sem0)

    plsc.subcore_barrier()

    @pl.when(c == 0)
    def _():
        _writeout(acc, cw, r0)

    @pl.when(c == 1)
    def _():
        _writeout(acc, cu, r0)


def _make_cnt():
    mesh = plsc.VectorSubcoreMesh(core_axis_name="c", subcore_axis_name="s")
    out_type = [jax.ShapeDtypeStruct((_NP, 128), jnp.float32)] * 2
    scratch = [
        pltpu.VMEM_SHARED((_NP, 128), jnp.float32),
        pltpu.VMEM((_IB, _B), jnp.int32),
        pltpu.VMEM((_B, 128), jnp.float32),
        pltpu.SemaphoreType.DMA,
    ]
    return pl.kernel(
        _cnt_body,
        out_type=out_type,
        mesh=mesh,
        scratch_types=scratch,
    )


def _agg2_body(pu, pw, s_uw, d_uw, s_wu, d_wu, zfeat,
               s2w, s2u,
               acc, rows, sidx, didx, sem0, sem1):
    c = lax.axis_index("c")
    s = lax.axis_index("s")
    r0 = s * _RT
    sems = (sem0, sem1)

    _zero_rows(zfeat, acc, r0)
    plsc.subcore_barrier()

    @pl.when(c == 0)
    def _():
        _gather_scatter_dir(s, pu, s_uw, d_uw, sidx, didx, rows, acc, sems)

    @pl.when(c == 1)
    def _():
        _gather_scatter_dir(s, pw, s_wu, d_wu, sidx, didx, rows, acc, sems)

    plsc.subcore_barrier()

    @pl.when(c == 0)
    def _():
        _writeout(acc, s2w, r0)

    @pl.when(c == 1)
    def _():
        _writeout(acc, s2u, r0)


def _make_agg2():
    mesh = plsc.VectorSubcoreMesh(core_axis_name="c", subcore_axis_name="s")
    out_type = [jax.ShapeDtypeStruct((_NP, 128), jnp.float32)] * 2
    scratch = [
        pltpu.VMEM_SHARED((_NP, 128), jnp.float32),
        pltpu.VMEM((2, _B, 128), jnp.float32),
        pltpu.VMEM((_IB, _B), jnp.int32),
        pltpu.VMEM((_IB, _B), jnp.int32),
        pltpu.SemaphoreType.DMA,
        pltpu.SemaphoreType.DMA,
    ]
    return pl.kernel(
        _agg2_body,
        out_type=out_type,
        mesh=mesh,
        scratch_types=scratch,
    )


def _comb_body(s0_ref, s1_ref, cnt_ref, h0_ref, h1_ref, l_ref, r_ref, b_ref,
               c2_ref, o_ref, p_ref):
    inv = 1.0 / jnp.maximum(cnt_ref[...][:, :1], 1.0)
    lw = l_ref[...]
    rw = r_ref[...]
    o = (_dot(s0_ref[...] * inv, lw[:128, :])
         + _dot(s1_ref[...] * inv, lw[128:, :])
         + _dot(h0_ref[...], rw[:128, :])
         + _dot(h1_ref[...], rw[128:, :])
         + b_ref[...])
    o = jnp.maximum(o, 0.0)
    o_ref[...] = o
    p = _dot(o, c2_ref[...])
    p_ref[...] = jnp.concatenate(
        [p, jnp.zeros((p.shape[0], 128 - _OUT), jnp.float32)], axis=1)


def _comb(s0, s1, cnt, h0, h1, lw, rw, b2, c2):
    return pl.pallas_call(
        _comb_body,
        grid=(_N // _BN,),
        in_specs=[
            pl.BlockSpec((_BN, 128), lambda i: (i, 0)),
            pl.BlockSpec((_BN, 128), lambda i: (i, 0)),
            pl.BlockSpec((_BN, 128), lambda i: (i, 0)),
            pl.BlockSpec((_BN, 128), lambda i: (i, 0)),
            pl.BlockSpec((_BN, 128), lambda i: (i, 0)),
            pl.BlockSpec((_HID, _HID), lambda i: (0, 0)),
            pl.BlockSpec((_HID, _HID), lambda i: (0, 0)),
            pl.BlockSpec((1, _HID), lambda i: (0, 0)),
            pl.BlockSpec((_HID, _OUT), lambda i: (0, 0)),
        ],
        out_specs=[
            pl.BlockSpec((_BN, _HID), lambda i: (i, 0)),
            pl.BlockSpec((_BN, 128), lambda i: (i, 0)),
        ],
        out_shape=[
            jax.ShapeDtypeStruct((_N, _HID), jnp.float32),
            jax.ShapeDtypeStruct((_N, 128), jnp.float32),
        ],
    )(s0, s1, cnt, h0, h1, lw, rw, b2, c2)


def _final_body(t_ref, cnt_ref, o_ref, r_ref, b_ref, z_ref):
    inv = 1.0 / jnp.maximum(cnt_ref[...][:, :1], 1.0)
    s2 = t_ref[...][:, :_OUT]
    z = s2 * inv + _dot(o_ref[...], r_ref[...]) + b_ref[...]
    m = jnp.max(z, axis=1, keepdims=True)
    ez = jnp.exp(z - m)
    lse = jnp.log(jnp.sum(ez, axis=1, keepdims=True))
    z_ref[...] = z - m - lse


def _final(t, cnt, o, rw, b2):
    return pl.pallas_call(
        _final_body,
        grid=(_N // _BN,),
        in_specs=[
            pl.BlockSpec((_BN, 128), lambda i: (i, 0)),
            pl.BlockSpec((_BN, 128), lambda i: (i, 0)),
            pl.BlockSpec((_BN, _HID), lambda i: (i, 0)),
            pl.BlockSpec((_HID, _OUT), lambda i: (0, 0)),
            pl.BlockSpec((1, _OUT), lambda i: (0, 0)),
        ],
        out_specs=pl.BlockSpec((_BN, _OUT), lambda i: (i, 0)),
        out_shape=jax.ShapeDtypeStruct((_N, _OUT), jnp.float32),
    )(t, cnt, o, rw, b2)


def kernel(x_websites, x_users, ei_u2w, ei_w2u,
           lin_w_web, lin_b_web, lin_w_usr, lin_b_usr,
           c1_uw_l, c1_uw_r, c1_uw_b, c1_wu_l, c1_wu_r, c1_wu_b,
           c2_uw_l, c2_uw_r, c2_uw_b, c2_wu_l, c2_wu_r, c2_wu_b):
    z128 = jnp.zeros((_NP, 128), jnp.float32)
    ones_h = jnp.ones((_B, 128), jnp.float32)
    s_uw = ei_u2w[0].reshape(_NT * _NB, _IB, _B)
    d_uw = ei_u2w[1].reshape(_NT * _NB, _IB, _B)
    s_wu = ei_w2u[0].reshape(_NT * _NB, _IB, _B)
    d_wu = ei_w2u[1].reshape(_NT * _NB, _IB, _B)

    cw, cu = _make_cnt()(d_uw, d_wu, z128, ones_h)

    h_w0, h_w1 = _proj(x_websites, lin_w_web, lin_b_web.reshape(1, -1))
    h_u0, h_u1 = _proj(x_users, lin_w_usr, lin_b_usr.reshape(1, -1))

    sw0, sw1, su0, su1 = _make_agg1()(
        h_u0, h_u1, h_w0, h_w1, s_uw, d_uw, s_wu, d_wu, z128)

    o_w, p_w = _comb(sw0, sw1, cw, h_w0, h_w1,
                     c1_uw_l, c1_uw_r, c1_uw_b.reshape(1, -1), c2_wu_l)
    o_u, p_u = _comb(su0, su1, cu, h_u0, h_u1,
                     c1_wu_l, c1_wu_r, c1_wu_b.reshape(1, -1), c2_uw_l)

    s2w, s2u = _make_agg2()(
        p_u, p_w, s_uw, d_uw, s_wu, d_wu, z128)

    z_w = _final(s2w, cw, o_w, c2_uw_r, c2_uw_b.reshape(1, -1))
    z_u = _final(s2u, cu, o_u, c2_wu_r, c2_wu_b.reshape(1, -1))
    return (z_w, z_u)
